# baseline (speedup 1.0000x reference)
import functools
import math

import jax
import jax.numpy as jnp
from jax import lax
from jax.experimental import pallas as pl
from jax.experimental.pallas import tpu as pltpu
from jax.experimental.pallas import tpu_sc as plsc

NC = 2
NS = 16
NW = NC * NS
L = 16

_MESH = plsc.VectorSubcoreMesh(core_axis_name="c", subcore_axis_name="s")


def _sc_gcn(n_pad, n_vec, nb, d, sb, eb, nbuf):
    rows_per_tile = n_pad // NS
    vec_per_tile = n_vec // NS
    fgroups = d // L
    n_stages = nb // sb

    @functools.partial(
        pl.kernel,
        out_type=[
            jax.ShapeDtypeStruct((n_pad, d), jnp.float32),
            jax.ShapeDtypeStruct((n_pad, d), jnp.float32),
        ],
        mesh=_MESH,
        compiler_params=pltpu.CompilerParams(needs_layout_passes=False),
        scratch_types=[
            pltpu.VMEM((sb, eb), jnp.int32),
            pltpu.VMEM((sb, eb), jnp.int32),
            pltpu.VMEM((sb, eb), jnp.float32),
            pltpu.VMEM((n_vec,), jnp.float32),
            pltpu.VMEM_SHARED((n_vec,), jnp.float32),
            pltpu.VMEM_SHARED((n_pad, d), jnp.float32),
        ]
        + [pltpu.VMEM((eb, d), jnp.float32) for _ in range(nbuf)]
        + [pltpu.SemaphoreType.DMA for _ in range(2 * nbuf)],
    )
    def gcn_sc(row_g, col_g, w_g, h_hbm, out0, out1,
               rowbuf, colbuf, wbuf, dinv_loc,
               deg_sh, acc, *bufs_sems):
        bufs = bufs_sems[:nbuf]
        gsems = bufs_sems[nbuf:2 * nbuf]
        ssems = bufs_sems[2 * nbuf:]
        cid = lax.axis_index("c")
        sid = lax.axis_index("s")
        wid = sid * NC + cid
        row_base = sid * rows_per_tile
        vec_base = sid * vec_per_tile
        rslice = pl.ds(row_base, rows_per_tile)
        zv = jnp.zeros((L,), jnp.float32)
        rows = bufs[0]

        def zbody(e, c):
            for f in range(fgroups):
                rows[e, pl.ds(f * L, L)] = zv
            return c
        lax.fori_loop(0, eb, zbody, 0)
        nz = rows_per_tile // eb
        for k in range(nz):
            pltpu.sync_copy(rows, acc.at[pl.ds(row_base + k * eb, eb)])
        rem = rows_per_tile % eb
        if rem:
            pltpu.sync_copy(rows.at[pl.ds(0, rem)],
                            acc.at[pl.ds(row_base + nz * eb, rem)])
        for k in range(vec_per_tile // eb):
            pltpu.sync_copy(rows.at[0, pl.ds(0, eb)],
                            deg_sh.at[pl.ds(vec_base + k * eb, eb)])
        vrem = vec_per_tile % eb
        if vrem:
            pltpu.sync_copy(
                rows.at[0, pl.ds(0, vrem)],
                deg_sh.at[pl.ds(vec_base + (vec_per_tile // eb) * eb,
                                vrem)])
        plsc.subcore_barrier()

        def dstage(stq, c):
            chunk = 2 * sid * n_stages + stq
            pltpu.sync_copy(col_g.at[chunk], colbuf)
            pltpu.sync_copy(w_g.at[chunk], wbuf)

            def dbody(j, c2):
                pltpu.sync_copy(wbuf.at[j], deg_sh.at[colbuf.at[j]],
                                add=True)
                return c2
            lax.fori_loop(0, sb, dbody, 0)
            return c
        lax.fori_loop(0, 2 * n_stages, dstage, 0)
        plsc.subcore_barrier()

        slicebuf = dinv_loc.at[pl.ds(0, vec_per_tile)]
        pltpu.sync_copy(deg_sh.at[pl.ds(vec_base, vec_per_tile)], slicebuf)

        def nbody(t, c):
            sl = pl.ds(t * L, L)
            xv = jnp.maximum(slicebuf[sl], 1e-12)
            s = jnp.maximum(xv, 1.0)
            for _ in range(15):
                s = 0.5 * (s + xv / s)
            slicebuf[sl] = s / xv
            return c
        lax.fori_loop(0, vec_per_tile // L, nbody, 0)
        pltpu.sync_copy(slicebuf, deg_sh.at[pl.ds(vec_base, vec_per_tile)])
        plsc.subcore_barrier()
        pltpu.sync_copy(deg_sh, dinv_loc)

        def scale_rows(buf, j):
            jv = jnp.broadcast_to(j, (L,)).astype(jnp.int32)

            def ebody(e, c3):
                ei = jnp.broadcast_to(e, (L,)).astype(jnp.int32)
                s = plsc.load_gather(wbuf, [jv, ei])
                for f in range(fgroups):
                    fs = pl.ds(f * L, L)
                    buf[e, fs] = buf[e, fs] * s
                return c3
            lax.fori_loop(0, eb, ebody, 0, unroll=4)

        def stage_body(st, c):
            chunk = wid * n_stages + st
            pltpu.sync_copy(row_g.at[chunk], rowbuf)
            pltpu.sync_copy(col_g.at[chunk], colbuf)
            pltpu.sync_copy(w_g.at[chunk], wbuf)

            for t in range(sb * eb // L):
                jj, gg = divmod(t, eb // L)
                sl = pl.ds(gg * L, L)
                rv = rowbuf[jj, sl]
                cv = colbuf[jj, sl]
                dr = plsc.load_gather(dinv_loc, [rv])
                dc = plsc.load_gather(dinv_loc, [cv])
                wbuf[jj, sl] = wbuf[jj, sl] * dr * dc

            lead = nbuf - 2
            gath = [None] * nbuf
            scat = [None] * nbuf
            for j in range(min(lead, sb)):
                gath[j] = pltpu.async_copy(
                    h_hbm.at[rowbuf.at[j]], bufs[j], gsems[j])
            for j in range(sb):
                p = j % nbuf
                jn = j + lead
                if jn < sb:
                    q = jn % nbuf
                    if scat[q] is not None:
                        scat[q].wait()
                    gath[q] = pltpu.async_copy(
                        h_hbm.at[rowbuf.at[jn]], bufs[q], gsems[q])
                gath[p].wait()
                scale_rows(bufs[p], j)
                scat[p] = pltpu.async_copy(
                    bufs[p], acc.at[colbuf.at[j]], ssems[p], add=True)
            for des in scat:
                if des is not None:
                    des.wait()
            return c
        lax.fori_loop(0, n_stages, stage_body, 0)
        plsc.subcore_barrier()

        @pl.when(cid == 0)
        def _():
            pltpu.sync_copy(acc.at[rslice], out0.at[rslice])

        @pl.when(cid == 1)
        def _():
            pltpu.sync_copy(acc.at[rslice], out1.at[rslice])

    return gcn_sc


def _tc_matmul(x_pad, W):
    n_pad, d_in = x_pad.shape
    d_out = W.shape[1]
    blk = 632
    return pl.pallas_call(
        lambda x_ref, w_ref, h_ref: h_ref.__setitem__(
            ..., jnp.dot(x_ref[...], w_ref[...],
                         preferred_element_type=jnp.float32)),
        grid=(n_pad // blk,),
        in_specs=[pl.BlockSpec((blk, d_in), lambda i: (i, 0)),
                  pl.BlockSpec((d_in, d_out), lambda i: (0, 0))],
        out_specs=pl.BlockSpec((blk, d_out), lambda i: (i, 0)),
        out_shape=jax.ShapeDtypeStruct((n_pad, d_out), jnp.float32),
    )(x_pad, W)


def _tc_combine(p0, p1, b2d, n, d):
    blk = 80
    return pl.pallas_call(
        lambda a_ref, b_ref, c_ref, o_ref: o_ref.__setitem__(
            ..., a_ref[...] + b_ref[...] + c_ref[...]),
        grid=(n // blk,),
        in_specs=[pl.BlockSpec((blk, d), lambda i: (i, 0)),
                  pl.BlockSpec((blk, d), lambda i: (i, 0)),
                  pl.BlockSpec((1, d), lambda i: (0, 0))],
        out_specs=pl.BlockSpec((blk, d), lambda i: (i, 0)),
        out_shape=jax.ShapeDtypeStruct((n, d), jnp.float32),
    )(p0, p1, b2d)


def kernel(x, edge_index, edge_weight, W, b):
    n, d_in = x.shape
    d = W.shape[1]
    e = edge_index.shape[1]

    eb = 96
    nbuf = 3
    sb_pref = 3
    tile_q = math.lcm(sb_pref * eb, eb)
    e_full = e + n
    per_chunk = -(-e_full // (NW * tile_q)) * tile_q
    e_pad = per_chunk * NW
    nb = per_chunk // eb
    pad = e_pad - e_full

    idx_dtype = edge_index.dtype
    loop_idx = jnp.arange(n, dtype=idx_dtype)
    zpad_i = jnp.zeros((pad,), dtype=idx_dtype)
    row_full = jnp.concatenate([edge_index[0], loop_idx, zpad_i])
    col_full = jnp.concatenate([edge_index[1], loop_idx, zpad_i])
    w_full = jnp.concatenate([edge_weight, jnp.ones((n,), jnp.float32),
                              jnp.zeros((pad,), jnp.float32)])

    n_pad = -(-n // (NS * 8)) * (NS * 8)
    n_vec = -(-n // (NS * L)) * (NS * L)

    sb = next(s for s in (sb_pref, 3, 2, 1) if nb % s == 0)
    n_stages = nb // sb
    row_s = row_full.reshape(NW * n_stages, sb, eb).astype(jnp.int32)
    col_s = col_full.reshape(NW * n_stages, sb, eb).astype(jnp.int32)
    w_s = w_full.reshape(NW * n_stages, sb, eb)
    x_pad = jnp.pad(x, ((0, n_pad - n), (0, 0)))

    h = _tc_matmul(x_pad, W)
    p0, p1 = _sc_gcn(n_pad, n_vec, nb, d, sb, eb, nbuf)(
        row_s, col_s, w_s, h)
    out = _tc_combine(p0, p1, b.reshape(1, d), n, d)
    return out
